# baseline (speedup 1.0000x reference)
import functools

import jax
import jax.numpy as jnp
import numpy as np
from jax import lax
from jax.experimental import pallas as pl
from jax.experimental.pallas import tpu as pltpu
from jax.experimental.pallas import tpu_sc as plsc

N = 10000
E = 320000
D = 128
NCF = 12
F = 128
GE = 256
OUT = 256
B = 100
MAXR = 15.0
IN_DIM = D - NCF + 3

NP = 10240
NC, NS = 2, 16
NW = NC * NS
E32 = E // NW
C = 64
NCHUNK = E32 // C
CT = E32 - NCHUNK * C
EPAD = E + 192
RPT = NP // NS
INVSQF = 1.0 / float(np.sqrt(F))


def _project(h, wq_ref, wk_ref, wv_ref, ws_ref,
             q_ref, k_ref, v_ref, s_ref, mq_ref, mk_ref, g):
    q = h @ wq_ref[...]
    k = h @ wk_ref[...]
    q_ref[...] = q
    k_ref[...] = k
    v_ref[...] = h @ wv_ref[...]
    s_ref[...] = h @ ws_ref[...]

    @pl.when(g == 0)
    def _():
        mq_ref[...] = jnp.zeros_like(mq_ref)
        mk_ref[...] = jnp.zeros_like(mk_ref)

    mq_ref[...] = jnp.maximum(mq_ref[...], jnp.max(jnp.sum(q * q, axis=1)))
    mk_ref[...] = jnp.maximum(mk_ref[...], jnp.max(jnp.sum(k * k, axis=1)))


def _embed_pre_body(xc_ref, w_ref, b_ref, wq_ref, wk_ref, wv_ref, ws_ref,
                    q_ref, k_ref, v_ref, s_ref, mq_ref, mk_ref):
    g = pl.program_id(0)
    h = jax.nn.gelu(xc_ref[...] @ w_ref[...] + b_ref[...])
    rid = g * 128 + lax.broadcasted_iota(jnp.int32, (128, 128), 0)
    h = jnp.where(rid < N, h, 0.0)
    _project(h, wq_ref, wk_ref, wv_ref, ws_ref,
             q_ref, k_ref, v_ref, s_ref, mq_ref, mk_ref, g)


def _ln_gelu(t):
    mu = jnp.mean(t, axis=-1, keepdims=True)
    var = jnp.mean((t - mu) ** 2, axis=-1, keepdims=True)
    return jax.nn.gelu((t - mu) / jnp.sqrt(var + 1e-5))


def _post_pre_body(s_ref, a0_ref, a1_ref, rd_ref,
                   wq_ref, wk_ref, wv_ref, ws_ref,
                   q_ref, k_ref, v_ref, so_ref, mq_ref, mk_ref):
    g = pl.program_id(0)
    h = _ln_gelu(s_ref[...] + (a0_ref[...] + a1_ref[...]) * rd_ref[...])
    _project(h, wq_ref, wk_ref, wv_ref, ws_ref,
             q_ref, k_ref, v_ref, so_ref, mq_ref, mk_ref, g)


def _post_pool_body(s_ref, a0_ref, a1_ref, rd_ref, wn_ref, bn_ref, o_ref):
    h = _ln_gelu(s_ref[...] + (a0_ref[...] + a1_ref[...]) * rd_ref[...])
    hn = h @ wn_ref[...] + bn_ref[...]
    o_ref[...] = jnp.max(hn.reshape(4, 100, 256), axis=1)[None]


def _mlp_body(cond_ref, w0_ref, b0_ref, wh_ref, bh_ref, wo_ref, bo_ref, out_ref):
    y = jax.nn.gelu(cond_ref[...] @ w0_ref[...] + b0_ref[...])
    for i in range(3):
        y = jax.nn.gelu(y @ wh_ref[i] + bh_ref[i])
    out_ref[...] = y @ wo_ref[...] + bo_ref[...]


def _sc_edge_body(q_hbm, k_hbm, v_hbm, src_hbm, dst_hbm, mg_hbm,
                  aggp_hbm, denp_hbm,
                  qA, qB, kA, kB, vS, ebuf, etail,
                  srcidxA, srcidxB, dstidxA, dstidxB, dstidxS,
                  dstidx16, mgbuf,
                  agg_sh, den_sh, semA, semB, semV, semIA, semIB, semS):
    cid = lax.axis_index("c")
    sid = lax.axis_index("s")
    wid = cid * NS + sid
    ebase = wid * E32

    pltpu.sync_copy(mg_hbm, mgbuf)

    def _zfill(i, carry):
        for fc in range(F // 16):
            qA[i, pl.ds(fc * 16, 16)] = jnp.zeros((16,), jnp.float32)
            vS[i, pl.ds(fc * 16, 16)] = jnp.zeros((16,), jnp.float32)
        return carry

    lax.fori_loop(0, C, _zfill, 0)
    for g in range(C // 16):
        ebuf[pl.ds(g * 16, 16)] = jnp.zeros((16,), jnp.float32)

    def _zagg(i, carry):
        pltpu.sync_copy(qA, agg_sh.at[pl.ds(sid * RPT + i * C, C)])
        pltpu.sync_copy(ebuf, den_sh.at[pl.ds(sid * RPT + i * C, C)])
        return carry

    lax.fori_loop(0, RPT // C, _zagg, 0)
    mgv = mgbuf[...]
    plsc.subcore_barrier()

    lane = lax.iota(jnp.int32, 16)

    gdn = lax.GatherDimensionNumbers(
        offset_dims=(), collapsed_slice_dims=(0,), start_index_map=(0,))

    def _take(vec, idx):
        return lax.gather(vec, idx[:, None], gdn, (1,),
                          mode=lax.GatherScatterMode.PROMISE_IN_BOUNDS)

    rots = [(lane + st) & 15 for st in (8, 4, 2, 1)]

    def _hsum(vec):
        for r in rots:
            vec = vec + _take(vec, r)
        return vec

    def fire_idx(ci, srcidx, dstidx, semI):
        off = ebase + ci * C
        pltpu.async_copy(src_hbm.at[pl.ds(off, C)], srcidx, semI)
        pltpu.async_copy(dst_hbm.at[pl.ds(off, C)], dstidx, semI)

    def drain_idx(srcidx, dstidx, semI):
        pltpu.make_async_copy(src_hbm.at[pl.ds(0, C)], srcidx, semI).wait()
        pltpu.make_async_copy(dst_hbm.at[pl.ds(0, C)], dstidx, semI).wait()

    def fire_qk(qbuf, kbuf, srcidx, dstidx, sem):
        pltpu.async_copy(q_hbm.at[dstidx], qbuf, sem)
        pltpu.async_copy(k_hbm.at[srcidx], kbuf, sem)

    def drain_qk(qbuf, kbuf, sem):
        pltpu.make_async_copy(q_hbm.at[pl.ds(0, C)], qbuf, sem).wait()
        pltpu.make_async_copy(k_hbm.at[pl.ds(0, C)], kbuf, sem).wait()

    def dot_stage(qbuf, kbuf, ebuf):
        def grp_body(gi, carry2):
            def edge_body(j, gv):
                i = gi * 16 + j
                acc = jnp.zeros((16,), jnp.float32)
                for fc in range(F // 16):
                    acc = acc + (qbuf[i, pl.ds(fc * 16, 16)]
                                 * kbuf[i, pl.ds(fc * 16, 16)])
                return jnp.where(lane == j, _hsum(acc), gv)

            gv = lax.fori_loop(0, 16, edge_body,
                               jnp.zeros((16,), jnp.float32), unroll=8)
            ebuf[pl.ds(gi * 16, 16)] = jnp.exp(gv * INVSQF - mgv)
            return carry2

        lax.fori_loop(0, C // 16, grp_body, 0)

    def ev_stage(vS, ebuf):
        def grp_body(gi, carry2):
            evec = ebuf[pl.ds(gi * 16, 16)]

            def edge_body(j, carry3):
                i = gi * 16 + j
                ev = _take(evec, jnp.full((16,), j, jnp.int32))
                for fc in range(F // 16):
                    vS[i, pl.ds(fc * 16, 16)] = (
                        ev * vS[i, pl.ds(fc * 16, 16)])
                return carry3

            lax.fori_loop(0, 16, edge_body, 0, unroll=4)
            return carry2

        lax.fori_loop(0, C // 16, grp_body, 0)

    def idx_copy(src, dst, n):
        for g in range(n // 16):
            dst[pl.ds(g * 16, 16)] = src[pl.ds(g * 16, 16)]

    def drain_scatter():
        pltpu.make_async_copy(vS, agg_sh.at[dstidxS], semS).wait()
        pltpu.make_async_copy(ebuf, den_sh.at[dstidxS], semS).wait()

    def half(ci, qX, kX, srcidxX, dstidxX, semX, semIX,
             qY, kY, srcidxY, dstidxY, semY, semIY):
        drain_idx(srcidxY, dstidxY, semIY)
        fire_qk(qY, kY, srcidxY, dstidxY, semY)
        drain_qk(qX, kX, semX)
        drain_scatter()
        pltpu.async_copy(v_hbm.at[srcidxX], vS, semV)
        dot_stage(qX, kX, ebuf)
        idx_copy(dstidxX, dstidxS, C)
        pltpu.make_async_copy(v_hbm.at[pl.ds(0, C)], vS, semV).wait()
        ev_stage(vS, ebuf)
        pltpu.async_copy(vS, agg_sh.at[dstidxS], semS, add=True)
        pltpu.async_copy(ebuf, den_sh.at[dstidxS], semS, add=True)
        fire_idx(ci + 2, srcidxX, dstidxX, semIX)

    pltpu.sync_copy(src_hbm.at[pl.ds(ebase, C)], srcidxA)
    pltpu.sync_copy(dst_hbm.at[pl.ds(ebase, C)], dstidxA)
    idx_copy(dstidxA, dstidxS, C)
    pltpu.async_copy(vS, agg_sh.at[dstidxS], semS, add=True)
    pltpu.async_copy(ebuf, den_sh.at[dstidxS], semS, add=True)
    fire_qk(qA, kA, srcidxA, dstidxA, semA)
    fire_idx(1, srcidxB, dstidxB, semIB)

    def loop_body(ii, carry):
        ci0 = ii * 2
        half(ci0, qA, kA, srcidxA, dstidxA, semA, semIA,
             qB, kB, srcidxB, dstidxB, semB, semIB)
        half(ci0 + 1, qB, kB, srcidxB, dstidxB, semB, semIB,
             qA, kA, srcidxA, dstidxA, semA, semIA)
        return carry

    lax.fori_loop(0, NCHUNK // 2, loop_body, 0)

    drain_qk(qA, kA, semA)
    drain_scatter()
    pltpu.async_copy(v_hbm.at[srcidxA], vS, semV)
    dot_stage(qA, kA, ebuf)
    pltpu.make_async_copy(v_hbm.at[pl.ds(0, C)], vS, semV).wait()
    ev_stage(vS, ebuf)
    etail[...] = ebuf[pl.ds(0, CT)]
    dstidx16[...] = dstidxA[pl.ds(0, CT)]
    pltpu.sync_copy(vS.at[pl.ds(0, CT)], agg_sh.at[dstidx16], add=True)
    pltpu.sync_copy(etail, den_sh.at[dstidx16], add=True)
    drain_idx(srcidxB, dstidxB, semIB)
    plsc.subcore_barrier()

    pltpu.sync_copy(agg_sh.at[pl.ds(sid * RPT, RPT)],
                    aggp_hbm.at[cid, pl.ds(sid * RPT, RPT)])
    pltpu.sync_copy(den_sh.at[pl.ds(sid * RPT, RPT)],
                    denp_hbm.at[cid, pl.ds(sid * RPT, RPT)])


_sc_edge = pl.kernel(
    _sc_edge_body,
    out_type=[
        jax.ShapeDtypeStruct((NC, NP, F), jnp.float32),
        jax.ShapeDtypeStruct((NC, NP), jnp.float32),
    ],
    mesh=plsc.VectorSubcoreMesh(core_axis_name="c", subcore_axis_name="s",
                                num_cores=NC, num_subcores=NS),
    scratch_types=[
        pltpu.VMEM((C, F), jnp.float32),
        pltpu.VMEM((C, F), jnp.float32),
        pltpu.VMEM((C, F), jnp.float32),
        pltpu.VMEM((C, F), jnp.float32),
        pltpu.VMEM((C, F), jnp.float32),
        pltpu.VMEM((C,), jnp.float32),
        pltpu.VMEM((CT,), jnp.float32),
        pltpu.VMEM((C,), jnp.int32),
        pltpu.VMEM((C,), jnp.int32),
        pltpu.VMEM((C,), jnp.int32),
        pltpu.VMEM((C,), jnp.int32),
        pltpu.VMEM((C,), jnp.int32),
        pltpu.VMEM((CT,), jnp.int32),
        pltpu.VMEM((16,), jnp.float32),
        pltpu.VMEM_SHARED((NP, F), jnp.float32),
        pltpu.VMEM_SHARED((NP,), jnp.float32),
        pltpu.SemaphoreType.DMA,
        pltpu.SemaphoreType.DMA,
        pltpu.SemaphoreType.DMA,
        pltpu.SemaphoreType.DMA,
        pltpu.SemaphoreType.DMA,
        pltpu.SemaphoreType.DMA,
    ],
)


def kernel(x, pos, edge_index, ptr, W_in, b_in, Wq, Wk, Wv, Ws, W_node,
           b_node, W_fc0, b_fc0, W_fch, b_fch, W_out, b_out):
    f32 = jnp.float32

    xcat = jnp.concatenate([x[:, :D - NCF], pos / MAXR], axis=-1)
    xcat = jnp.pad(xcat, ((0, NP - N), (0, 128 - IN_DIM)))
    w_cat = jnp.pad(W_in, ((0, 128 - IN_DIM), (0, 0)))

    _wspec = pl.BlockSpec((F, F), lambda g: (0, 0))
    _nspec = pl.BlockSpec((128, F), lambda g: (g, 0))
    _qkvs_out = dict(
        out_specs=[
            _nspec, _nspec, _nspec, _nspec,
            pl.BlockSpec((8, 128), lambda g: (0, 0)),
            pl.BlockSpec((8, 128), lambda g: (0, 0)),
        ],
        out_shape=[
            jax.ShapeDtypeStruct((NP, F), f32),
            jax.ShapeDtypeStruct((NP, F), f32),
            jax.ShapeDtypeStruct((NP, F), f32),
            jax.ShapeDtypeStruct((NP, F), f32),
            jax.ShapeDtypeStruct((8, 128), f32),
            jax.ShapeDtypeStruct((8, 128), f32),
        ],
    )

    q, k, v, s, mq, mk = pl.pallas_call(
        _embed_pre_body,
        grid=(NP // 128,),
        in_specs=[_nspec, _wspec, pl.BlockSpec((1, F), lambda g: (0, 0)),
                  _wspec, _wspec, _wspec, _wspec],
        **_qkvs_out,
    )(xcat, w_cat, b_in[None], Wq[0], Wk[0], Wv[0], Ws[0])

    post_pre = pl.pallas_call(
        _post_pre_body,
        grid=(NP // 128,),
        in_specs=[_nspec, _nspec, _nspec,
                  pl.BlockSpec((128, 1), lambda g: (g, 0)),
                  _wspec, _wspec, _wspec, _wspec],
        **_qkvs_out,
    )

    srcp = jnp.pad(edge_index[0], (0, EPAD - E))
    dstp = jnp.pad(edge_index[1], (0, EPAD - E))
    for l in range(4):
        mg = jnp.sqrt(jnp.max(mq)) * jnp.sqrt(jnp.max(mk)) * INVSQF
        mg16 = jnp.full((16,), mg, f32)
        aggp, denp = _sc_edge(q, k, v, srcp, dstp, mg16)
        den = denp[0] + denp[1]
        rd = jnp.where(den > 0, 1.0 / den, 0.0)[:, None]
        if l < 3:
            q, k, v, s, mq, mk = post_pre(s, aggp[0], aggp[1], rd,
                                          Wq[l + 1], Wk[l + 1],
                                          Wv[l + 1], Ws[l + 1])

    pooled = pl.pallas_call(
        _post_pool_body,
        grid=(25,),
        in_specs=[
            pl.BlockSpec((400, 128), lambda g: (g, 0)),
            pl.BlockSpec((400, 128), lambda g: (g, 0)),
            pl.BlockSpec((400, 128), lambda g: (g, 0)),
            pl.BlockSpec((400, 1), lambda g: (g, 0)),
            pl.BlockSpec((F, GE), lambda g: (0, 0)),
            pl.BlockSpec((1, GE), lambda g: (0, 0)),
        ],
        out_specs=pl.BlockSpec((1, 4, GE), lambda g: (g, 0, 0)),
        out_shape=jax.ShapeDtypeStruct((25, 4, GE), f32),
    )(s, aggp[0], aggp[1], rd, W_node, b_node[None]).reshape(B, GE)

    crys = x[ptr[:-1], -NCF:]
    cond = jnp.concatenate([pooled, crys], axis=-1)
    cond_p = jnp.zeros((128, 384), f32).at[:B, :GE + NCF].set(cond)
    w0_p = jnp.zeros((384, 256), f32).at[:GE + NCF].set(W_fc0)
    out = pl.pallas_call(
        _mlp_body,
        out_shape=jax.ShapeDtypeStruct((128, OUT), f32),
    )(cond_p, w0_p, b_fc0, W_fch, b_fch, W_out, b_out)
    return out[:B]


# speedup vs baseline: 1.0022x; 1.0022x over previous
import functools

import jax
import jax.numpy as jnp
import numpy as np
from jax import lax
from jax.experimental import pallas as pl
from jax.experimental.pallas import tpu as pltpu
from jax.experimental.pallas import tpu_sc as plsc

N = 10000
E = 320000
D = 128
NCF = 12
F = 128
GE = 256
OUT = 256
B = 100
MAXR = 15.0
IN_DIM = D - NCF + 3

NP = 10240
NC, NS = 2, 16
NW = NC * NS
E32 = E // NW
C = 64
NCHUNK = E32 // C
CT = E32 - NCHUNK * C
EPAD = E + 192
RPT = NP // NS
INVSQF = 1.0 / float(np.sqrt(F))


def _project(h, wq_ref, wk_ref, wv_ref, ws_ref,
             q_ref, k_ref, v_ref, s_ref, mq_ref, mk_ref, g):
    q = h @ wq_ref[...]
    k = h @ wk_ref[...]
    q_ref[...] = q
    k_ref[...] = k
    v_ref[...] = h @ wv_ref[...]
    s_ref[...] = h @ ws_ref[...]

    @pl.when(g == 0)
    def _():
        mq_ref[...] = jnp.zeros_like(mq_ref)
        mk_ref[...] = jnp.zeros_like(mk_ref)

    mq_ref[...] = jnp.maximum(mq_ref[...], jnp.max(jnp.sum(q * q, axis=1)))
    mk_ref[...] = jnp.maximum(mk_ref[...], jnp.max(jnp.sum(k * k, axis=1)))


def _embed_pre_body(xc_ref, w_ref, b_ref, wq_ref, wk_ref, wv_ref, ws_ref,
                    q_ref, k_ref, v_ref, s_ref, mq_ref, mk_ref):
    g = pl.program_id(0)
    h = jax.nn.gelu(xc_ref[...] @ w_ref[...] + b_ref[...])
    rid = g * 128 + lax.broadcasted_iota(jnp.int32, (128, 128), 0)
    h = jnp.where(rid < N, h, 0.0)
    _project(h, wq_ref, wk_ref, wv_ref, ws_ref,
             q_ref, k_ref, v_ref, s_ref, mq_ref, mk_ref, g)


def _ln_gelu(t):
    mu = jnp.mean(t, axis=-1, keepdims=True)
    var = jnp.mean((t - mu) ** 2, axis=-1, keepdims=True)
    return jax.nn.gelu((t - mu) / jnp.sqrt(var + 1e-5))


def _post_pre_body(s_ref, a0_ref, a1_ref, rd_ref,
                   wq_ref, wk_ref, wv_ref, ws_ref,
                   q_ref, k_ref, v_ref, so_ref, mq_ref, mk_ref):
    g = pl.program_id(0)
    h = _ln_gelu(s_ref[...] + (a0_ref[...] + a1_ref[...]) * rd_ref[...])
    _project(h, wq_ref, wk_ref, wv_ref, ws_ref,
             q_ref, k_ref, v_ref, so_ref, mq_ref, mk_ref, g)


def _post_pool_body(s_ref, a0_ref, a1_ref, rd_ref, wn_ref, bn_ref, o_ref):
    h = _ln_gelu(s_ref[...] + (a0_ref[...] + a1_ref[...]) * rd_ref[...])
    hn = h @ wn_ref[...] + bn_ref[...]
    o_ref[...] = jnp.max(hn.reshape(4, 100, 256), axis=1)[None]


def _mlp_body(cond_ref, w0_ref, b0_ref, wh_ref, bh_ref, wo_ref, bo_ref, out_ref):
    y = jax.nn.gelu(cond_ref[...] @ w0_ref[...] + b0_ref[...])
    for i in range(3):
        y = jax.nn.gelu(y @ wh_ref[i] + bh_ref[i])
    out_ref[...] = y @ wo_ref[...] + bo_ref[...]


def _sc_edge_body(q_hbm, k_hbm, v_hbm, src_hbm, dst_hbm, mg_hbm,
                  aggp_hbm, denp_hbm,
                  qA, qB, kA, kB, vS, ebuf, etail,
                  srcidxA, srcidxB, dstidxA, dstidxB, dstidxS,
                  dstidx16, mgbuf,
                  agg_sh, den_sh, semA, semB, semV, semIA, semIB, semS):
    cid = lax.axis_index("c")
    sid = lax.axis_index("s")
    wid = cid * NS + sid
    ebase = wid * E32

    pltpu.sync_copy(mg_hbm, mgbuf)

    def _zfill(i, carry):
        for fc in range(F // 16):
            qA[i, pl.ds(fc * 16, 16)] = jnp.zeros((16,), jnp.float32)
            vS[i, pl.ds(fc * 16, 16)] = jnp.zeros((16,), jnp.float32)
        return carry

    lax.fori_loop(0, C, _zfill, 0)
    for g in range(C // 16):
        ebuf[pl.ds(g * 16, 16)] = jnp.zeros((16,), jnp.float32)

    def _zagg(i, carry):
        pltpu.sync_copy(qA, agg_sh.at[pl.ds(sid * RPT + i * C, C)])
        pltpu.sync_copy(ebuf, den_sh.at[pl.ds(sid * RPT + i * C, C)])
        return carry

    lax.fori_loop(0, RPT // C, _zagg, 0)
    mgv = mgbuf[...]
    plsc.subcore_barrier()

    lane = lax.iota(jnp.int32, 16)

    gdn = lax.GatherDimensionNumbers(
        offset_dims=(), collapsed_slice_dims=(0,), start_index_map=(0,))

    def _take(vec, idx):
        return lax.gather(vec, idx[:, None], gdn, (1,),
                          mode=lax.GatherScatterMode.PROMISE_IN_BOUNDS)

    rots = [(lane + st) & 15 for st in (8, 4, 2, 1)]

    def _hsum(vec):
        for r in rots:
            vec = vec + _take(vec, r)
        return vec

    def fire_idx(ci, srcidx, dstidx, semI):
        off = ebase + ci * C
        pltpu.async_copy(src_hbm.at[pl.ds(off, C)], srcidx, semI)
        pltpu.async_copy(dst_hbm.at[pl.ds(off, C)], dstidx, semI)

    def drain_idx(srcidx, dstidx, semI):
        pltpu.make_async_copy(src_hbm.at[pl.ds(0, C)], srcidx, semI).wait()
        pltpu.make_async_copy(dst_hbm.at[pl.ds(0, C)], dstidx, semI).wait()

    def fire_qk(qbuf, kbuf, srcidx, dstidx, sem):
        pltpu.async_copy(q_hbm.at[dstidx], qbuf, sem)
        pltpu.async_copy(k_hbm.at[srcidx], kbuf, sem)

    def drain_qk(qbuf, kbuf, sem):
        pltpu.make_async_copy(q_hbm.at[pl.ds(0, C)], qbuf, sem).wait()
        pltpu.make_async_copy(k_hbm.at[pl.ds(0, C)], kbuf, sem).wait()

    def dot_stage(qbuf, kbuf, ebuf):
        @plsc.parallel_loop(0, C // 16)
        def _(gi):
            def edge_body(j, gv):
                i = gi * 16 + j
                acc = jnp.zeros((16,), jnp.float32)
                for fc in range(F // 16):
                    acc = acc + (qbuf[i, pl.ds(fc * 16, 16)]
                                 * kbuf[i, pl.ds(fc * 16, 16)])
                return jnp.where(lane == j, _hsum(acc), gv)

            gv = lax.fori_loop(0, 16, edge_body,
                               jnp.zeros((16,), jnp.float32), unroll=4)
            ebuf[pl.ds(gi * 16, 16)] = jnp.exp(gv * INVSQF - mgv)

    def ev_stage(vS, ebuf):
        @plsc.parallel_loop(0, C // 16)
        def _(gi):
            evec = ebuf[pl.ds(gi * 16, 16)]

            def edge_body(j, carry3):
                i = gi * 16 + j
                ev = _take(evec, jnp.full((16,), j, jnp.int32))
                for fc in range(F // 16):
                    vS[i, pl.ds(fc * 16, 16)] = (
                        ev * vS[i, pl.ds(fc * 16, 16)])
                return carry3

            lax.fori_loop(0, 16, edge_body, 0, unroll=4)

    def idx_copy(src, dst, n):
        for g in range(n // 16):
            dst[pl.ds(g * 16, 16)] = src[pl.ds(g * 16, 16)]

    def drain_scatter():
        pltpu.make_async_copy(vS, agg_sh.at[dstidxS], semS).wait()
        pltpu.make_async_copy(ebuf, den_sh.at[dstidxS], semS).wait()

    def half(ci, qX, kX, srcidxX, dstidxX, semX, semIX,
             qY, kY, srcidxY, dstidxY, semY, semIY):
        drain_idx(srcidxY, dstidxY, semIY)
        fire_qk(qY, kY, srcidxY, dstidxY, semY)
        drain_qk(qX, kX, semX)
        drain_scatter()
        pltpu.async_copy(v_hbm.at[srcidxX], vS, semV)
        dot_stage(qX, kX, ebuf)
        idx_copy(dstidxX, dstidxS, C)
        pltpu.make_async_copy(v_hbm.at[pl.ds(0, C)], vS, semV).wait()
        ev_stage(vS, ebuf)
        pltpu.async_copy(vS, agg_sh.at[dstidxS], semS, add=True)
        pltpu.async_copy(ebuf, den_sh.at[dstidxS], semS, add=True)
        fire_idx(ci + 2, srcidxX, dstidxX, semIX)

    pltpu.sync_copy(src_hbm.at[pl.ds(ebase, C)], srcidxA)
    pltpu.sync_copy(dst_hbm.at[pl.ds(ebase, C)], dstidxA)
    idx_copy(dstidxA, dstidxS, C)
    pltpu.async_copy(vS, agg_sh.at[dstidxS], semS, add=True)
    pltpu.async_copy(ebuf, den_sh.at[dstidxS], semS, add=True)
    fire_qk(qA, kA, srcidxA, dstidxA, semA)
    fire_idx(1, srcidxB, dstidxB, semIB)

    def loop_body(ii, carry):
        ci0 = ii * 2
        half(ci0, qA, kA, srcidxA, dstidxA, semA, semIA,
             qB, kB, srcidxB, dstidxB, semB, semIB)
        half(ci0 + 1, qB, kB, srcidxB, dstidxB, semB, semIB,
             qA, kA, srcidxA, dstidxA, semA, semIA)
        return carry

    lax.fori_loop(0, NCHUNK // 2, loop_body, 0)

    drain_qk(qA, kA, semA)
    drain_scatter()
    pltpu.async_copy(v_hbm.at[srcidxA], vS, semV)
    dot_stage(qA, kA, ebuf)
    pltpu.make_async_copy(v_hbm.at[pl.ds(0, C)], vS, semV).wait()
    ev_stage(vS, ebuf)
    etail[...] = ebuf[pl.ds(0, CT)]
    dstidx16[...] = dstidxA[pl.ds(0, CT)]
    pltpu.sync_copy(vS.at[pl.ds(0, CT)], agg_sh.at[dstidx16], add=True)
    pltpu.sync_copy(etail, den_sh.at[dstidx16], add=True)
    drain_idx(srcidxB, dstidxB, semIB)
    plsc.subcore_barrier()

    pltpu.sync_copy(agg_sh.at[pl.ds(sid * RPT, RPT)],
                    aggp_hbm.at[cid, pl.ds(sid * RPT, RPT)])
    pltpu.sync_copy(den_sh.at[pl.ds(sid * RPT, RPT)],
                    denp_hbm.at[cid, pl.ds(sid * RPT, RPT)])


_sc_edge = pl.kernel(
    _sc_edge_body,
    out_type=[
        jax.ShapeDtypeStruct((NC, NP, F), jnp.float32),
        jax.ShapeDtypeStruct((NC, NP), jnp.float32),
    ],
    mesh=plsc.VectorSubcoreMesh(core_axis_name="c", subcore_axis_name="s",
                                num_cores=NC, num_subcores=NS),
    scratch_types=[
        pltpu.VMEM((C, F), jnp.float32),
        pltpu.VMEM((C, F), jnp.float32),
        pltpu.VMEM((C, F), jnp.float32),
        pltpu.VMEM((C, F), jnp.float32),
        pltpu.VMEM((C, F), jnp.float32),
        pltpu.VMEM((C,), jnp.float32),
        pltpu.VMEM((CT,), jnp.float32),
        pltpu.VMEM((C,), jnp.int32),
        pltpu.VMEM((C,), jnp.int32),
        pltpu.VMEM((C,), jnp.int32),
        pltpu.VMEM((C,), jnp.int32),
        pltpu.VMEM((C,), jnp.int32),
        pltpu.VMEM((CT,), jnp.int32),
        pltpu.VMEM((16,), jnp.float32),
        pltpu.VMEM_SHARED((NP, F), jnp.float32),
        pltpu.VMEM_SHARED((NP,), jnp.float32),
        pltpu.SemaphoreType.DMA,
        pltpu.SemaphoreType.DMA,
        pltpu.SemaphoreType.DMA,
        pltpu.SemaphoreType.DMA,
        pltpu.SemaphoreType.DMA,
        pltpu.SemaphoreType.DMA,
    ],
)


def kernel(x, pos, edge_index, ptr, W_in, b_in, Wq, Wk, Wv, Ws, W_node,
           b_node, W_fc0, b_fc0, W_fch, b_fch, W_out, b_out):
    f32 = jnp.float32

    xcat = jnp.concatenate([x[:, :D - NCF], pos / MAXR], axis=-1)
    xcat = jnp.pad(xcat, ((0, NP - N), (0, 128 - IN_DIM)))
    w_cat = jnp.pad(W_in, ((0, 128 - IN_DIM), (0, 0)))

    _wspec = pl.BlockSpec((F, F), lambda g: (0, 0))
    _nspec = pl.BlockSpec((128, F), lambda g: (g, 0))
    _qkvs_out = dict(
        out_specs=[
            _nspec, _nspec, _nspec, _nspec,
            pl.BlockSpec((8, 128), lambda g: (0, 0)),
            pl.BlockSpec((8, 128), lambda g: (0, 0)),
        ],
        out_shape=[
            jax.ShapeDtypeStruct((NP, F), f32),
            jax.ShapeDtypeStruct((NP, F), f32),
            jax.ShapeDtypeStruct((NP, F), f32),
            jax.ShapeDtypeStruct((NP, F), f32),
            jax.ShapeDtypeStruct((8, 128), f32),
            jax.ShapeDtypeStruct((8, 128), f32),
        ],
    )

    q, k, v, s, mq, mk = pl.pallas_call(
        _embed_pre_body,
        grid=(NP // 128,),
        in_specs=[_nspec, _wspec, pl.BlockSpec((1, F), lambda g: (0, 0)),
                  _wspec, _wspec, _wspec, _wspec],
        **_qkvs_out,
    )(xcat, w_cat, b_in[None], Wq[0], Wk[0], Wv[0], Ws[0])

    post_pre = pl.pallas_call(
        _post_pre_body,
        grid=(NP // 128,),
        in_specs=[_nspec, _nspec, _nspec,
                  pl.BlockSpec((128, 1), lambda g: (g, 0)),
                  _wspec, _wspec, _wspec, _wspec],
        **_qkvs_out,
    )

    srcp = jnp.pad(edge_index[0], (0, EPAD - E))
    dstp = jnp.pad(edge_index[1], (0, EPAD - E))
    for l in range(4):
        mg = jnp.sqrt(jnp.max(mq)) * jnp.sqrt(jnp.max(mk)) * INVSQF
        mg16 = jnp.full((16,), mg, f32)
        aggp, denp = _sc_edge(q, k, v, srcp, dstp, mg16)
        den = denp[0] + denp[1]
        rd = jnp.where(den > 0, 1.0 / den, 0.0)[:, None]
        if l < 3:
            q, k, v, s, mq, mk = post_pre(s, aggp[0], aggp[1], rd,
                                          Wq[l + 1], Wk[l + 1],
                                          Wv[l + 1], Ws[l + 1])

    pooled = pl.pallas_call(
        _post_pool_body,
        grid=(25,),
        in_specs=[
            pl.BlockSpec((400, 128), lambda g: (g, 0)),
            pl.BlockSpec((400, 128), lambda g: (g, 0)),
            pl.BlockSpec((400, 128), lambda g: (g, 0)),
            pl.BlockSpec((400, 1), lambda g: (g, 0)),
            pl.BlockSpec((F, GE), lambda g: (0, 0)),
            pl.BlockSpec((1, GE), lambda g: (0, 0)),
        ],
        out_specs=pl.BlockSpec((1, 4, GE), lambda g: (g, 0, 0)),
        out_shape=jax.ShapeDtypeStruct((25, 4, GE), f32),
    )(s, aggp[0], aggp[1], rd, W_node, b_node[None]).reshape(B, GE)

    crys = x[ptr[:-1], -NCF:]
    cond = jnp.concatenate([pooled, crys], axis=-1)
    cond_p = jnp.zeros((128, 384), f32).at[:B, :GE + NCF].set(cond)
    w0_p = jnp.zeros((384, 256), f32).at[:GE + NCF].set(W_fc0)
    out = pl.pallas_call(
        _mlp_body,
        out_shape=jax.ShapeDtypeStruct((128, OUT), f32),
    )(cond_p, w0_p, b_fc0, W_fch, b_fch, W_out, b_out)
    return out[:B]


# speedup vs baseline: 1.1629x; 1.1603x over previous
import functools

import jax
import jax.numpy as jnp
import numpy as np
from jax import lax
from jax.experimental import pallas as pl
from jax.experimental.pallas import tpu as pltpu
from jax.experimental.pallas import tpu_sc as plsc

N = 10000
E = 320000
D = 128
NCF = 12
F = 128
GE = 256
OUT = 256
B = 100
MAXR = 15.0
IN_DIM = D - NCF + 3

NP = 10240
NC, NS = 2, 16
NW = NC * NS
E32 = E // NW
C = 80
NCHUNK = E32 // C
EPAD = E + 192
RPT = NP // NS
INVSQF = 1.0 / float(np.sqrt(F))


def _project(h, wq_ref, wk_ref, wv_ref, ws_ref,
             q_ref, k_ref, v_ref, s_ref, mq_ref, mk_ref, g):
    q = h @ wq_ref[...]
    k = h @ wk_ref[...]
    q_ref[...] = q
    k_ref[...] = k
    v_ref[...] = h @ wv_ref[...]
    s_ref[...] = h @ ws_ref[...]

    @pl.when(g == 0)
    def _():
        mq_ref[...] = jnp.zeros_like(mq_ref)
        mk_ref[...] = jnp.zeros_like(mk_ref)

    mq_ref[...] = jnp.maximum(mq_ref[...], jnp.max(jnp.sum(q * q, axis=1)))
    mk_ref[...] = jnp.maximum(mk_ref[...], jnp.max(jnp.sum(k * k, axis=1)))


def _embed_pre_body(xc_ref, w_ref, b_ref, wq_ref, wk_ref, wv_ref, ws_ref,
                    q_ref, k_ref, v_ref, s_ref, mq_ref, mk_ref):
    g = pl.program_id(0)
    h = jax.nn.gelu(xc_ref[...] @ w_ref[...] + b_ref[...])
    rid = g * 128 + lax.broadcasted_iota(jnp.int32, (128, 128), 0)
    h = jnp.where(rid < N, h, 0.0)
    _project(h, wq_ref, wk_ref, wv_ref, ws_ref,
             q_ref, k_ref, v_ref, s_ref, mq_ref, mk_ref, g)


def _ln_gelu(t):
    mu = jnp.mean(t, axis=-1, keepdims=True)
    var = jnp.mean((t - mu) ** 2, axis=-1, keepdims=True)
    return jax.nn.gelu((t - mu) / jnp.sqrt(var + 1e-5))


def _post_pre_body(s_ref, a0_ref, a1_ref, rd_ref,
                   wq_ref, wk_ref, wv_ref, ws_ref,
                   q_ref, k_ref, v_ref, so_ref, mq_ref, mk_ref):
    g = pl.program_id(0)
    h = _ln_gelu(s_ref[...] + (a0_ref[...] + a1_ref[...]) * rd_ref[...])
    _project(h, wq_ref, wk_ref, wv_ref, ws_ref,
             q_ref, k_ref, v_ref, so_ref, mq_ref, mk_ref, g)


def _post_pool_body(s_ref, a0_ref, a1_ref, rd_ref, wn_ref, bn_ref, o_ref):
    h = _ln_gelu(s_ref[...] + (a0_ref[...] + a1_ref[...]) * rd_ref[...])
    hn = h @ wn_ref[...] + bn_ref[...]
    o_ref[...] = jnp.max(hn.reshape(4, 100, 256), axis=1)[None]


def _mlp_body(cond_ref, w0_ref, b0_ref, wh_ref, bh_ref, wo_ref, bo_ref, out_ref):
    y = jax.nn.gelu(cond_ref[...] @ w0_ref[...] + b0_ref[...])
    for i in range(3):
        y = jax.nn.gelu(y @ wh_ref[i] + bh_ref[i])
    out_ref[...] = y @ wo_ref[...] + bo_ref[...]


def _sc_edge_body(q_hbm, k_hbm, v_hbm, src_hbm, dst_hbm, mg_hbm,
                  aggp_hbm, denp_hbm,
                  qS, kA, kB, vS, ebuf,
                  srcidxA, srcidxB, dstidxA, dstidxB, dstidxS, mgbuf,
                  agg_sh, den_sh, semA, semB, semQ, semV, semIA, semIB, semS):
    cid = lax.axis_index("c")
    sid = lax.axis_index("s")
    wid = cid * NS + sid
    ebase = wid * E32

    pltpu.sync_copy(mg_hbm, mgbuf)

    def _zfill(i, carry):
        for fc in range(F // 16):
            qS[i, pl.ds(fc * 16, 16)] = jnp.zeros((16,), jnp.float32)
            vS[i, pl.ds(fc * 16, 16)] = jnp.zeros((16,), jnp.float32)
        return carry

    lax.fori_loop(0, C, _zfill, 0)
    for g in range(C // 16):
        ebuf[pl.ds(g * 16, 16)] = jnp.zeros((16,), jnp.float32)

    def _zagg(i, carry):
        pltpu.sync_copy(qS, agg_sh.at[pl.ds(sid * RPT + i * C, C)])
        pltpu.sync_copy(ebuf, den_sh.at[pl.ds(sid * RPT + i * C, C)])
        return carry

    lax.fori_loop(0, RPT // C, _zagg, 0)
    mgv = mgbuf[...]
    plsc.subcore_barrier()

    lane = lax.iota(jnp.int32, 16)

    gdn = lax.GatherDimensionNumbers(
        offset_dims=(), collapsed_slice_dims=(0,), start_index_map=(0,))

    def _take(vec, idx):
        return lax.gather(vec, idx[:, None], gdn, (1,),
                          mode=lax.GatherScatterMode.PROMISE_IN_BOUNDS)

    rots = [(lane + st) & 15 for st in (8, 4, 2, 1)]

    def _hsum(vec):
        for r in rots:
            vec = vec + _take(vec, r)
        return vec

    def fire_idx(ci, srcidx, dstidx, semI):
        off = ebase + ci * C
        pltpu.async_copy(src_hbm.at[pl.ds(off, C)], srcidx, semI)
        pltpu.async_copy(dst_hbm.at[pl.ds(off, C)], dstidx, semI)

    def drain_idx(srcidx, dstidx, semI):
        pltpu.make_async_copy(src_hbm.at[pl.ds(0, C)], srcidx, semI).wait()
        pltpu.make_async_copy(dst_hbm.at[pl.ds(0, C)], dstidx, semI).wait()

    def idx_copy(src, dst, n):
        for g in range(n // 16):
            dst[pl.ds(g * 16, 16)] = src[pl.ds(g * 16, 16)]

    def dot_stage(qbuf, kbuf, eb):
        @plsc.parallel_loop(0, C // 16)
        def _(gi):
            def edge_body(j, gv):
                i = gi * 16 + j
                acc = jnp.zeros((16,), jnp.float32)
                for fc in range(F // 16):
                    acc = acc + (qbuf[i, pl.ds(fc * 16, 16)]
                                 * kbuf[i, pl.ds(fc * 16, 16)])
                return jnp.where(lane == j, _hsum(acc), gv)

            gv = lax.fori_loop(0, 16, edge_body,
                               jnp.zeros((16,), jnp.float32), unroll=4)
            eb[pl.ds(gi * 16, 16)] = jnp.exp(gv * INVSQF - mgv)

    def ev_stage(vb, eb):
        @plsc.parallel_loop(0, C // 16)
        def _(gi):
            evec = eb[pl.ds(gi * 16, 16)]

            def edge_body(j, carry3):
                i = gi * 16 + j
                ev = _take(evec, jnp.full((16,), j, jnp.int32))
                for fc in range(F // 16):
                    vb[i, pl.ds(fc * 16, 16)] = (
                        ev * vb[i, pl.ds(fc * 16, 16)])
                return carry3

            lax.fori_loop(0, 16, edge_body, 0, unroll=4)

    def drain_scatter():
        pltpu.make_async_copy(vS, agg_sh.at[dstidxS], semS).wait()
        pltpu.make_async_copy(ebuf, den_sh.at[dstidxS], semS).wait()

    def half(ci, kX, srcidxX, dstidxX, semX, semIX,
             kY, srcidxY, dstidxY, semY, semIY):
        drain_idx(srcidxY, dstidxY, semIY)
        pltpu.async_copy(k_hbm.at[srcidxY], kY, semY)
        pltpu.make_async_copy(k_hbm.at[pl.ds(0, C)], kX, semX).wait()
        pltpu.make_async_copy(q_hbm.at[pl.ds(0, C)], qS, semQ).wait()
        drain_scatter()
        pltpu.async_copy(v_hbm.at[srcidxX], vS, semV)
        dot_stage(qS, kX, ebuf)
        pltpu.async_copy(q_hbm.at[dstidxY], qS, semQ)
        idx_copy(dstidxX, dstidxS, C)
        pltpu.make_async_copy(v_hbm.at[pl.ds(0, C)], vS, semV).wait()
        ev_stage(vS, ebuf)
        pltpu.async_copy(vS, agg_sh.at[dstidxS], semS, add=True)
        pltpu.async_copy(ebuf, den_sh.at[dstidxS], semS, add=True)
        fire_idx(ci + 2, srcidxX, dstidxX, semIX)

    pltpu.sync_copy(src_hbm.at[pl.ds(ebase, C)], srcidxA)
    pltpu.sync_copy(dst_hbm.at[pl.ds(ebase, C)], dstidxA)
    idx_copy(dstidxA, dstidxS, C)
    pltpu.async_copy(vS, agg_sh.at[dstidxS], semS, add=True)
    pltpu.async_copy(ebuf, den_sh.at[dstidxS], semS, add=True)
    pltpu.async_copy(k_hbm.at[srcidxA], kA, semA)
    pltpu.async_copy(q_hbm.at[dstidxA], qS, semQ)
    fire_idx(1, srcidxB, dstidxB, semIB)

    def loop_body(ii, carry):
        ci0 = ii * 2
        half(ci0, kA, srcidxA, dstidxA, semA, semIA,
             kB, srcidxB, dstidxB, semB, semIB)
        half(ci0 + 1, kB, srcidxB, dstidxB, semB, semIB,
             kA, srcidxA, dstidxA, semA, semIA)
        return carry

    lax.fori_loop(0, NCHUNK // 2, loop_body, 0)

    pltpu.make_async_copy(k_hbm.at[pl.ds(0, C)], kA, semA).wait()
    pltpu.make_async_copy(q_hbm.at[pl.ds(0, C)], qS, semQ).wait()
    drain_scatter()
    pltpu.async_copy(v_hbm.at[srcidxA], vS, semV)
    dot_stage(qS, kA, ebuf)
    idx_copy(dstidxA, dstidxS, C)
    pltpu.make_async_copy(v_hbm.at[pl.ds(0, C)], vS, semV).wait()
    ev_stage(vS, ebuf)
    pltpu.sync_copy(vS, agg_sh.at[dstidxS], add=True)
    pltpu.sync_copy(ebuf, den_sh.at[dstidxS], add=True)
    drain_idx(srcidxB, dstidxB, semIB)
    plsc.subcore_barrier()

    pltpu.sync_copy(agg_sh.at[pl.ds(sid * RPT, RPT)],
                    aggp_hbm.at[cid, pl.ds(sid * RPT, RPT)])
    pltpu.sync_copy(den_sh.at[pl.ds(sid * RPT, RPT)],
                    denp_hbm.at[cid, pl.ds(sid * RPT, RPT)])


_sc_edge = pl.kernel(
    _sc_edge_body,
    out_type=[
        jax.ShapeDtypeStruct((NC, NP, F), jnp.float32),
        jax.ShapeDtypeStruct((NC, NP), jnp.float32),
    ],
    mesh=plsc.VectorSubcoreMesh(core_axis_name="c", subcore_axis_name="s",
                                num_cores=NC, num_subcores=NS),
    scratch_types=[
        pltpu.VMEM((C, F), jnp.float32),
        pltpu.VMEM((C, F), jnp.float32),
        pltpu.VMEM((C, F), jnp.float32),
        pltpu.VMEM((C, F), jnp.float32),
        pltpu.VMEM((C,), jnp.float32),
        pltpu.VMEM((C,), jnp.int32),
        pltpu.VMEM((C,), jnp.int32),
        pltpu.VMEM((C,), jnp.int32),
        pltpu.VMEM((C,), jnp.int32),
        pltpu.VMEM((C,), jnp.int32),
        pltpu.VMEM((16,), jnp.float32),
        pltpu.VMEM_SHARED((NP, F), jnp.float32),
        pltpu.VMEM_SHARED((NP,), jnp.float32),
        pltpu.SemaphoreType.DMA,
        pltpu.SemaphoreType.DMA,
        pltpu.SemaphoreType.DMA,
        pltpu.SemaphoreType.DMA,
        pltpu.SemaphoreType.DMA,
        pltpu.SemaphoreType.DMA,
        pltpu.SemaphoreType.DMA,
    ],
)


def kernel(x, pos, edge_index, ptr, W_in, b_in, Wq, Wk, Wv, Ws, W_node,
           b_node, W_fc0, b_fc0, W_fch, b_fch, W_out, b_out):
    f32 = jnp.float32

    xcat = jnp.concatenate([x[:, :D - NCF], pos / MAXR], axis=-1)
    xcat = jnp.pad(xcat, ((0, NP - N), (0, 128 - IN_DIM)))
    w_cat = jnp.pad(W_in, ((0, 128 - IN_DIM), (0, 0)))

    _wspec = pl.BlockSpec((F, F), lambda g: (0, 0))
    _nspec = pl.BlockSpec((128, F), lambda g: (g, 0))
    _qkvs_out = dict(
        out_specs=[
            _nspec, _nspec, _nspec, _nspec,
            pl.BlockSpec((8, 128), lambda g: (0, 0)),
            pl.BlockSpec((8, 128), lambda g: (0, 0)),
        ],
        out_shape=[
            jax.ShapeDtypeStruct((NP, F), f32),
            jax.ShapeDtypeStruct((NP, F), f32),
            jax.ShapeDtypeStruct((NP, F), f32),
            jax.ShapeDtypeStruct((NP, F), f32),
            jax.ShapeDtypeStruct((8, 128), f32),
            jax.ShapeDtypeStruct((8, 128), f32),
        ],
    )

    q, k, v, s, mq, mk = pl.pallas_call(
        _embed_pre_body,
        grid=(NP // 128,),
        in_specs=[_nspec, _wspec, pl.BlockSpec((1, F), lambda g: (0, 0)),
                  _wspec, _wspec, _wspec, _wspec],
        **_qkvs_out,
    )(xcat, w_cat, b_in[None], Wq[0], Wk[0], Wv[0], Ws[0])

    post_pre = pl.pallas_call(
        _post_pre_body,
        grid=(NP // 128,),
        in_specs=[_nspec, _nspec, _nspec,
                  pl.BlockSpec((128, 1), lambda g: (g, 0)),
                  _wspec, _wspec, _wspec, _wspec],
        **_qkvs_out,
    )

    srcp = jnp.pad(edge_index[0], (0, EPAD - E))
    dstp = jnp.pad(edge_index[1], (0, EPAD - E))
    for l in range(4):
        mg = jnp.sqrt(jnp.max(mq)) * jnp.sqrt(jnp.max(mk)) * INVSQF
        mg16 = jnp.full((16,), mg, f32)
        aggp, denp = _sc_edge(q, k, v, srcp, dstp, mg16)
        den = denp[0] + denp[1]
        rd = jnp.where(den > 0, 1.0 / den, 0.0)[:, None]
        if l < 3:
            q, k, v, s, mq, mk = post_pre(s, aggp[0], aggp[1], rd,
                                          Wq[l + 1], Wk[l + 1],
                                          Wv[l + 1], Ws[l + 1])

    pooled = pl.pallas_call(
        _post_pool_body,
        grid=(25,),
        in_specs=[
            pl.BlockSpec((400, 128), lambda g: (g, 0)),
            pl.BlockSpec((400, 128), lambda g: (g, 0)),
            pl.BlockSpec((400, 128), lambda g: (g, 0)),
            pl.BlockSpec((400, 1), lambda g: (g, 0)),
            pl.BlockSpec((F, GE), lambda g: (0, 0)),
            pl.BlockSpec((1, GE), lambda g: (0, 0)),
        ],
        out_specs=pl.BlockSpec((1, 4, GE), lambda g: (g, 0, 0)),
        out_shape=jax.ShapeDtypeStruct((25, 4, GE), f32),
    )(s, aggp[0], aggp[1], rd, W_node, b_node[None]).reshape(B, GE)

    crys = x[ptr[:-1], -NCF:]
    cond = jnp.concatenate([pooled, crys], axis=-1)
    cond_p = jnp.zeros((128, 384), f32).at[:B, :GE + NCF].set(cond)
    w0_p = jnp.zeros((384, 256), f32).at[:GE + NCF].set(W_fc0)
    out = pl.pallas_call(
        _mlp_body,
        out_shape=jax.ShapeDtypeStruct((128, OUT), f32),
    )(cond_p, w0_p, b_fc0, W_fch, b_fch, W_out, b_out)
    return out[:B]
